# baseline (speedup 1.0000x reference)
import jax
import jax.numpy as jnp
from jax import lax
from jax.experimental import pallas as pl
from jax.experimental.pallas import tpu as pltpu
from jax.experimental.pallas import tpu_sc as plsc

_NC = 2
_NS = 16
_NW = _NC * _NS
_CHUNK = 128
_PACK = 4


def _sc_gather_body(uidx, iidx, gu_t, gi_t, mu_t, mi_t,
                    out_gu, out_gi, out_mu, out_mi,
                    uv, iv, guv, giv, muv, miv, sem):
    ch = uv.shape[0]
    bpw = ch * _CHUNK
    wid = lax.axis_index("c") * _NS + lax.axis_index("s")
    base = wid * bpw
    pltpu.sync_copy(uidx.at[wid], uv)
    pltpu.sync_copy(iidx.at[wid], iv)

    def chunk(c):
        copies = []
        for tbl, idxv, dst in ((gu_t, uv, guv), (gi_t, iv, giv),
                               (mu_t, uv, muv), (mi_t, iv, miv)):
            copies.append(pltpu.async_copy(tbl.at[idxv.at[c]], dst, sem))
        for cp in copies:
            cp.wait()
        copies = []
        for out, src in ((out_gu, guv), (out_gi, giv),
                         (out_mu, muv), (out_mi, miv)):
            copies.append(pltpu.async_copy(
                src, out.at[pl.ds(base + c * _CHUNK, _CHUNK)], sem))
        for cp in copies:
            cp.wait()

    pl.loop(0, ch)(chunk)


def _sc_gather(uidx, iidx, gu_t, gi_t, mu_t, mi_t):
    batch = uidx.shape[0] * uidx.shape[1] * uidx.shape[2]
    ch = uidx.shape[1]
    row = jax.ShapeDtypeStruct((batch, 128), jnp.float32)
    gather = pl.kernel(
        _sc_gather_body,
        out_type=(row, row, row, row),
        mesh=plsc.VectorSubcoreMesh(core_axis_name="c", subcore_axis_name="s"),
        scratch_types=[
            pltpu.VMEM((ch, _CHUNK), jnp.int32),
            pltpu.VMEM((ch, _CHUNK), jnp.int32),
            pltpu.VMEM((_CHUNK, 128), jnp.float32),
            pltpu.VMEM((_CHUNK, 128), jnp.float32),
            pltpu.VMEM((_CHUNK, 128), jnp.float32),
            pltpu.VMEM((_CHUNK, 128), jnp.float32),
            pltpu.SemaphoreType.DMA,
        ],
        compiler_params=pltpu.CompilerParams(use_tc_tiling_on_sc=True),
    )
    return gather(uidx, iidx, gu_t, gi_t, mu_t, mi_t)


def _select(rows, onehot, dim):
    acc = onehot[:, 0:1] * rows[:, :dim]
    for k in range(1, _PACK):
        acc += onehot[:, k:k + 1] * rows[:, k * dim:(k + 1) * dim]
    return acc


def _mlp_body(gu_ref, gi_ref, mu_ref, mi_ref, ohu_ref, ohi_ref,
              w1a_ref, w1b_ref, b1_ref, w2_ref, b2_ref, w3_ref, b3_ref,
              wog_ref, woh_ref, bo_ref, out_ref):
    f32 = jnp.float32
    ohu = ohu_ref[...]
    ohi = ohi_ref[...]
    mu = _select(mu_ref[...], ohu, 32)
    mi = _select(mi_ref[...], ohi, 32)
    gmf = _select(gu_ref[...], ohu, 32) * _select(gi_ref[...], ohi, 32)
    h = jnp.dot(mu, w1a_ref[...], preferred_element_type=f32)
    h += jnp.dot(mi, w1b_ref[...], preferred_element_type=f32)
    h = jnp.maximum(h + b1_ref[...], 0.0)
    h = jnp.maximum(
        jnp.dot(h, w2_ref[...], preferred_element_type=f32) + b2_ref[...], 0.0)
    h = jnp.maximum(
        jnp.dot(h, w3_ref[...], preferred_element_type=f32) + b3_ref[...], 0.0)
    logit = (jnp.sum(gmf * wog_ref[...], axis=1)
             + jnp.sum(h * woh_ref[...], axis=1) + bo_ref[0, 0])
    out_ref[...] = jax.nn.sigmoid(logit)


def _mlp(gu, gi, mu, mi, ohu, ohi, W1, b1, W2, b2, W3, b3, Wo, bo,
         block_rows):
    batch = mu.shape[0]
    mdim = W1.shape[0] // 2
    gdim = Wo.shape[0] - W3.shape[1]
    w1a = W1[:mdim]
    w1b = W1[mdim:]
    wog = Wo[:gdim].reshape(1, gdim)
    woh = Wo[gdim:].reshape(1, Wo.shape[0] - gdim)
    row_spec = pl.BlockSpec((block_rows, 128), lambda i: (i, 0))
    oh_spec = pl.BlockSpec((block_rows, _PACK), lambda i: (i, 0))
    full = lambda a: pl.BlockSpec(a.shape, lambda i: (0,) * a.ndim)
    args = (gu, gi, mu, mi, ohu, ohi, w1a, w1b, b1.reshape(1, -1), W2,
            b2.reshape(1, -1), W3, b3.reshape(1, -1), wog, woh,
            bo.reshape(1, 1))
    in_specs = [row_spec] * 4 + [oh_spec] * 2 + [full(a) for a in args[6:]]
    return pl.pallas_call(
        _mlp_body,
        grid=(batch // block_rows,),
        in_specs=in_specs,
        out_specs=pl.BlockSpec((block_rows,), lambda i: (i,)),
        out_shape=jax.ShapeDtypeStruct((batch,), jnp.float32),
    )(*args)


def kernel(user_indices, item_indices, gmf_user_table, gmf_item_table,
           mlp_user_table, mlp_item_table, W1, b1, W2, b2, W3, b3, Wo, bo):
    batch = user_indices.shape[0]
    ch = batch // (_NW * _CHUNK)
    uidx = (user_indices // _PACK).reshape(_NW, ch, _CHUNK)
    iidx = (item_indices // _PACK).reshape(_NW, ch, _CHUNK)
    ohu = jax.nn.one_hot(user_indices % _PACK, _PACK, dtype=jnp.float32)
    ohi = jax.nn.one_hot(item_indices % _PACK, _PACK, dtype=jnp.float32)
    packed = lambda t: t.reshape(t.shape[0] * t.shape[1] // 128, 128)
    gu, gi, mu, mi = _sc_gather(uidx, iidx, packed(gmf_user_table),
                                packed(gmf_item_table),
                                packed(mlp_user_table),
                                packed(mlp_item_table))
    return _mlp(gu, gi, mu, mi, ohu, ohi, W1, b1, W2, b2, W3, b3, Wo, bo,
                block_rows=2048)


# speedup vs baseline: 4.2906x; 4.2906x over previous
import jax
import jax.numpy as jnp
from jax import lax
from jax.experimental import pallas as pl
from jax.experimental.pallas import tpu as pltpu
from jax.experimental.pallas import tpu_sc as plsc

_NC = 2
_NS = 16
_NW = _NC * _NS
_CHUNK = 128
_PACK = 4


def _sc_gather_body(idx, tbl, out, iv, buf0, buf1, sem):
    ch = iv.shape[0]
    bpw = ch * _CHUNK
    wid = lax.axis_index("c") * _NS + lax.axis_index("s")
    base = wid * bpw
    pltpu.sync_copy(idx.at[wid], iv)
    bufs = (buf0, buf1)
    gathers = [pltpu.async_copy(tbl.at[iv.at[0]], buf0, sem)]
    for c in range(1, ch + 1):
        if c < ch:
            gathers.append(
                pltpu.async_copy(tbl.at[iv.at[c]], bufs[c % 2], sem))
        gathers[c - 1].wait()
        pltpu.sync_copy(bufs[(c - 1) % 2],
                        out.at[pl.ds(base + (c - 1) * _CHUNK, _CHUNK)])


def _sc_gather_one(idx, tbl, batch):
    ch = idx.shape[1]
    gather = pl.kernel(
        _sc_gather_body,
        out_type=jax.ShapeDtypeStruct((batch, 128), jnp.float32),
        mesh=plsc.VectorSubcoreMesh(core_axis_name="c", subcore_axis_name="s"),
        scratch_types=[
            pltpu.VMEM((ch, _CHUNK), jnp.int32),
            pltpu.VMEM((_CHUNK, 128), jnp.float32),
            pltpu.VMEM((_CHUNK, 128), jnp.float32),
            pltpu.SemaphoreType.DMA,
        ],
        compiler_params=pltpu.CompilerParams(use_tc_tiling_on_sc=True),
    )
    return gather(idx, tbl)


_BLK_U = 65536
_BRES = _BLK_U // _PACK


def _repack_body(t_ref, eye_ref, out_ref):
    x = t_ref[...]
    eye = eye_ref[...]
    xr = jnp.concatenate(
        [x[:, k * _BRES:(k + 1) * _BRES] for k in range(_PACK)], axis=0)
    out_ref[...] = lax.dot_general(xr, eye, (((0,), (0,)), ((), ())),
                                   preferred_element_type=jnp.float32)


def _repack(tT, eye):
    dim, vocab = tT.shape
    grid = (vocab + _BLK_U - 1) // _BLK_U
    return pl.pallas_call(
        _repack_body,
        grid=(grid,),
        in_specs=[pl.BlockSpec((dim, _BLK_U), lambda i: (0, i)),
                  pl.BlockSpec((128, 128), lambda i: (0, 0))],
        out_specs=pl.BlockSpec((_BRES, 128), lambda i: (i, 0)),
        out_shape=jax.ShapeDtypeStruct((grid * _BRES, 128), jnp.float32),
    )(tT, eye)


def _mlp_body(gu_ref, gi_ref, mu_ref, mi_ref, ohu_ref, ohi_ref,
              e4_ref, w1su_ref, w1si_ref, b1_ref, w2_ref, b2_ref,
              w3_ref, b3_ref, p_ref, wog_ref, woh_ref, bo_ref, out_ref):
    f32 = jnp.float32
    dot = lambda a, b: jnp.dot(a, b, preferred_element_type=f32)
    mask_u = dot(ohu_ref[...], e4_ref[...])
    mask_i = dot(ohi_ref[...], e4_ref[...])
    h = dot(mu_ref[...] * mask_u, w1su_ref[...])
    h += dot(mi_ref[...] * mask_i, w1si_ref[...])
    h = jnp.maximum(h + b1_ref[...], 0.0)
    h = jnp.maximum(dot(h, w2_ref[...]) + b2_ref[...], 0.0)
    h = jnp.maximum(dot(h, w3_ref[...]) + b3_ref[...], 0.0)
    gmf = dot(gu_ref[...] * mask_u, p_ref[...]) * dot(
        gi_ref[...] * mask_i, p_ref[...])
    logit = dot(gmf, wog_ref[...]) + dot(h, woh_ref[...]) + bo_ref[0, 0]
    out_ref[...] = jax.nn.sigmoid(logit)[:, 0]


def _mlp(gu, gi, mu, mi, ohu, ohi, W1, b1, W2, b2, W3, b3, Wo, bo,
         block_rows):
    batch = mu.shape[0]
    mdim = W1.shape[0] // 2
    gdim = Wo.shape[0] - W3.shape[1]
    w1a = W1[:mdim]
    w1b = W1[mdim:]
    wog = Wo[:gdim]
    woh = Wo[gdim:]
    e4 = jnp.repeat(jnp.eye(_PACK, dtype=jnp.float32), mdim, axis=1)
    w1su = jnp.tile(w1a, (_PACK, 1))
    w1si = jnp.tile(w1b, (_PACK, 1))
    psel = jnp.tile(jnp.eye(gdim, dtype=jnp.float32), (_PACK, 1))
    row_spec = pl.BlockSpec((block_rows, 128), lambda i: (i, 0))
    oh_spec = pl.BlockSpec((block_rows, _PACK), lambda i: (i, 0))
    full = lambda a: pl.BlockSpec(a.shape, lambda i: (0,) * a.ndim)
    args = (gu, gi, mu, mi, ohu, ohi, e4, w1su, w1si, b1.reshape(1, -1),
            W2, b2.reshape(1, -1), W3, b3.reshape(1, -1), psel, wog, woh,
            bo.reshape(1, 1))
    in_specs = [row_spec] * 4 + [oh_spec] * 2 + [full(a) for a in args[6:]]
    return pl.pallas_call(
        _mlp_body,
        grid=(batch // block_rows,),
        in_specs=in_specs,
        out_specs=pl.BlockSpec((block_rows,), lambda i: (i,)),
        out_shape=jax.ShapeDtypeStruct((batch,), jnp.float32),
    )(*args)


def kernel(user_indices, item_indices, gmf_user_table, gmf_item_table,
           mlp_user_table, mlp_item_table, W1, b1, W2, b2, W3, b3, Wo, bo):
    batch = user_indices.shape[0]
    ch = batch // (_NW * _CHUNK)
    urow = (user_indices >> 16) * _BRES + (user_indices & (_BRES - 1))
    irow = (item_indices >> 16) * _BRES + (item_indices & (_BRES - 1))
    uidx = urow.reshape(_NW, ch, _CHUNK)
    iidx = irow.reshape(_NW, ch, _CHUNK)
    ohu = jax.nn.one_hot((user_indices >> 14) & 3, _PACK, dtype=jnp.float32)
    ohi = jax.nn.one_hot((item_indices >> 14) & 3, _PACK, dtype=jnp.float32)
    eye = jnp.eye(128, dtype=jnp.float32)
    packed = lambda t: _repack(t.T, eye)
    gu = _sc_gather_one(uidx, packed(gmf_user_table), batch)
    gi = _sc_gather_one(iidx, packed(gmf_item_table), batch)
    mu = _sc_gather_one(uidx, packed(mlp_user_table), batch)
    mi = _sc_gather_one(iidx, packed(mlp_item_table), batch)
    return _mlp(gu, gi, mu, mi, ohu, ohi, W1, b1, W2, b2, W3, b3, Wo, bo,
                block_rows=2048)
